# baseline (speedup 1.0000x reference)
import jax
import jax.numpy as jnp
from jax import lax
from jax.experimental import pallas as pl
from jax.experimental.pallas import tpu as pltpu
from jax.experimental.pallas import tpu_sc as plsc

BATCH = 4096
SEQ = 50
EMBED = 64
LANES = 16
NGROUP = EMBED // LANES
SEQP = 64

NC, NS = 2, 16
NW = NC * NS
ROWS_W = BATCH // NW
CHUNK_TOK = 2 * SEQP
NCHUNK = ROWS_W // 2
TROWS = BATCH * SEQP // CHUNK_TOK


def _body(idx_hbm, mask_hbm, table_hbm, out_hbm, idx_v, mask_v, idxp0, idxp1,
          rows0, rows1, out_v, sem0, sem1):
    wid = lax.axis_index("s") * NC + lax.axis_index("c")
    rows_b = (rows0, rows1)
    idxp_b = (idxp0, idxp1)
    sems = (sem0, sem1)

    pltpu.sync_copy(idx_hbm.at[pl.ds(wid * NCHUNK, NCHUNK), :], idx_v)
    pltpu.sync_copy(mask_hbm.at[pl.ds(wid * NCHUNK, NCHUNK), :], mask_v)

    def prep_and_start(g, b):
        for t in range(CHUNK_TOK // LANES):
            idxp_b[b][pl.ds(t * LANES, LANES)] = (
                idx_v[g, pl.ds(t * LANES, LANES)] >> 1)
        pltpu.make_async_copy(
            table_hbm.at[idxp_b[b]], rows_b[b], sems[b]).start()

    def compute(g, b):
        rows = rows_b[b]
        for j in range(2):
            base = SEQP * j
            mv = [mask_v[g, pl.ds(base + t * LANES, LANES)] for t in range(4)]
            iv = [idx_v[g, pl.ds(base + t * LANES, LANES)] for t in range(4)]
            acc = [jnp.zeros((LANES,), jnp.float32) for _ in range(NGROUP)]
            for s in range(SEQ):
                m = mv[s // LANES][s % LANES]
                off = (iv[s // LANES][s % LANES] & 1) << 6
                r = base + s
                for k in range(NGROUP):
                    acc[k] = acc[k] + rows[r, pl.ds(off + k * LANES, LANES)] * m
            denom = jnp.broadcast_to(
                jnp.maximum(jnp.sum(mv[0] + mv[1] + mv[2] + mv[3]), 1.0),
                (LANES,))
            for k in range(NGROUP):
                out_v[g, pl.ds(base + k * LANES, LANES)] = acc[k] / denom

    prep_and_start(0, 0)
    prep_and_start(1, 1)

    def tb(t, carry):
        for b in range(2):
            g = 2 * t + b
            pltpu.make_async_copy(
                table_hbm.at[idxp_b[b]], rows_b[b], sems[b]).wait()
            compute(g, b)

            @pl.when(g + 2 < NCHUNK)
            def _():
                prep_and_start(g + 2, b)
        return carry

    lax.fori_loop(0, NCHUNK // 2, tb, 0)

    pltpu.sync_copy(out_v, out_hbm.at[pl.ds(wid * NCHUNK, NCHUNK), :])


@jax.jit
def _embed(idx2, mask2, table2):
    mesh = plsc.VectorSubcoreMesh(core_axis_name="c", subcore_axis_name="s")
    f = pl.kernel(
        _body,
        out_type=jax.ShapeDtypeStruct((TROWS, CHUNK_TOK), jnp.float32),
        mesh=mesh,
        scratch_types=[
            pltpu.VMEM((NCHUNK, CHUNK_TOK), jnp.int32),
            pltpu.VMEM((NCHUNK, CHUNK_TOK), jnp.float32),
            pltpu.VMEM((CHUNK_TOK,), jnp.int32),
            pltpu.VMEM((CHUNK_TOK,), jnp.int32),
            pltpu.VMEM((CHUNK_TOK, 2 * EMBED), jnp.float32),
            pltpu.VMEM((CHUNK_TOK, 2 * EMBED), jnp.float32),
            pltpu.VMEM((NCHUNK, CHUNK_TOK), jnp.float32),
            pltpu.SemaphoreType.DMA,
            pltpu.SemaphoreType.DMA,
        ],
        compiler_params=pltpu.CompilerParams(needs_layout_passes=False),
    )
    return f(idx2, mask2, table2)


def kernel(token_indices, mask, embedding_table):
    idxp = jnp.pad(token_indices, ((0, 0), (0, SEQP - SEQ)))
    maskp = jnp.pad(mask, ((0, 0), (0, SEQP - SEQ)))
    idx2 = idxp.reshape(TROWS, CHUNK_TOK)
    mask2 = maskp.reshape(TROWS, CHUNK_TOK)
    table2 = embedding_table.reshape(-1, 2 * EMBED)
    out2 = _embed(idx2, mask2, table2)
    return out2.reshape(BATCH, EMBED)


# speedup vs baseline: 1.3858x; 1.3858x over previous
import jax
import jax.numpy as jnp
from jax import lax
from jax.experimental import pallas as pl
from jax.experimental.pallas import tpu as pltpu
from jax.experimental.pallas import tpu_sc as plsc

BATCH = 4096
SEQ = 50
EMBED = 64
PADW = 128
LANES = 16
NGROUP = EMBED // LANES

NC, NS = 2, 16
NW = NC * NS
ROWS_W = BATCH // NW
CB = 2
CHUNK_TOK = CB * SEQ
NCHUNK = ROWS_W // CB
TOK_W = ROWS_W * SEQ
NBUF = 4


def _body(idx_hbm, mask_hbm, table_hbm, out_hbm, idx_v, mask_v, rows0, rows1,
          rows2, rows3, out_v, sem0, sem1, sem2, sem3):
    wid = lax.axis_index("s") * NC + lax.axis_index("c")
    rows_b = (rows0, rows1, rows2, rows3)
    sems = (sem0, sem1, sem2, sem3)

    pltpu.sync_copy(idx_hbm.at[pl.ds(wid * NCHUNK, NCHUNK), :], idx_v)
    pltpu.sync_copy(mask_hbm.at[pl.ds(wid * TOK_W, TOK_W)],
                    mask_v.at[pl.ds(0, TOK_W)])

    def start(g, b):
        pltpu.make_async_copy(
            table_hbm.at[idx_v.at[g]], rows_b[b], sems[b]).start()

    def compute(g, b):
        rows = rows_b[b]
        zero = jnp.zeros((LANES,), jnp.float32)

        def s_step(s, acc_all):
            new = []
            for j in range(CB):
                a = acc_all[j]
                tok = g * CHUNK_TOK + j * SEQ + s
                m = mask_v[pl.ds(tok, LANES)][0]
                r = j * SEQ + s
                vals = [a[k] + rows[r, pl.ds(k * LANES, LANES)] * m
                        for k in range(NGROUP)]
                vals.append(a[NGROUP] + m)
                new.append(tuple(vals))
            return tuple(new)

        init = tuple(tuple(zero for _ in range(NGROUP)) + (jnp.float32(0.0),)
                     for _ in range(CB))
        acc_all = lax.fori_loop(0, SEQ, s_step, init)
        for j in range(CB):
            denom = jnp.broadcast_to(
                jnp.maximum(acc_all[j][NGROUP], 1.0), (LANES,))
            for k in range(NGROUP):
                out_v[g * CB + j, pl.ds(k * LANES, LANES)] = (
                    acc_all[j][k] / denom)

    for b in range(NBUF):
        start(b, b)

    def tb(t, carry):
        for b in range(NBUF):
            g = NBUF * t + b
            pltpu.make_async_copy(
                table_hbm.at[idx_v.at[g]], rows_b[b], sems[b]).wait()
            compute(g, b)

            @pl.when(g + NBUF < NCHUNK)
            def _():
                start(g + NBUF, b)
        return carry

    lax.fori_loop(0, NCHUNK // NBUF, tb, 0)

    pltpu.sync_copy(out_v, out_hbm.at[pl.ds(wid * ROWS_W, ROWS_W), :])


@jax.jit
def _embed(idx2, maskf, table2):
    mesh = plsc.VectorSubcoreMesh(core_axis_name="c", subcore_axis_name="s")
    f = pl.kernel(
        _body,
        out_type=jax.ShapeDtypeStruct((BATCH, EMBED), jnp.float32),
        mesh=mesh,
        scratch_types=[
            pltpu.VMEM((NCHUNK, CHUNK_TOK), jnp.int32),
            pltpu.VMEM((TOK_W + LANES,), jnp.float32),
            pltpu.VMEM((CHUNK_TOK, PADW), jnp.float32),
            pltpu.VMEM((CHUNK_TOK, PADW), jnp.float32),
            pltpu.VMEM((CHUNK_TOK, PADW), jnp.float32),
            pltpu.VMEM((CHUNK_TOK, PADW), jnp.float32),
            pltpu.VMEM((ROWS_W, EMBED), jnp.float32),
            pltpu.SemaphoreType.DMA,
            pltpu.SemaphoreType.DMA,
            pltpu.SemaphoreType.DMA,
            pltpu.SemaphoreType.DMA,
        ],
        compiler_params=pltpu.CompilerParams(use_tc_tiling_on_sc=False),
    )
    return f(idx2, maskf, table2)


def kernel(token_indices, mask, embedding_table):
    idx2 = token_indices.reshape(BATCH // CB, CHUNK_TOK)
    maskf = mask.reshape(-1)
    table2 = jnp.pad(embedding_table, ((0, 0), (0, PADW - EMBED)))
    return _embed(idx2, maskf, table2)
